# baseline (speedup 1.0000x reference)
import functools

import jax
import jax.numpy as jnp
from jax import lax
from jax.experimental import pallas as pl
from jax.experimental.pallas import tpu as pltpu
from jax.experimental.pallas import tpu_sc as plsc

N = 10000
E = 640000
NW = 32
EW = E // NW
CH = 10000
G_HID = 32
N_OUT = 8
HID = 64
HD2 = 2 * HID

_MESH = dict(core_axis_name="c", subcore_axis_name="s", num_cores=2,
             num_subcores=16)


def _sc_scatter1(h_hbm, src_hbm, dst_hbm, out_hbm, h_v, acc_v, src_v, dst_v):
    cid = lax.axis_index("c")
    sid = lax.axis_index("s")
    wid = sid * 2 + cid
    base = wid * EW
    pltpu.sync_copy(h_hbm, h_v)

    zeros = jnp.zeros((16,), jnp.float32)

    def zbody(i, carry):
        acc_v[pl.ds(pl.multiple_of(i * 16, 16), 16)] = zeros
        return carry

    lax.fori_loop(0, N // 16, zbody, 0)

    def chunk(ci, carry):
        pltpu.sync_copy(src_hbm.at[pl.ds(base + ci * CH, CH)], src_v)
        pltpu.sync_copy(dst_hbm.at[pl.ds(base + ci * CH, CH)], dst_v)

        def vec(i, c2):
            off = pl.multiple_of(i * 16, 16)
            si = src_v[pl.ds(off, 16)]
            di = dst_v[pl.ds(off, 16)]
            vals = plsc.load_gather(h_v, [si])
            plsc.addupdate_scatter(acc_v, [di], vals)
            return c2

        lax.fori_loop(0, CH // 16, vec, 0)
        return carry

    lax.fori_loop(0, EW // CH, chunk, 0)
    pltpu.sync_copy(acc_v, out_hbm.at[wid])


_scatter1 = functools.partial(
    pl.kernel,
    out_type=jax.ShapeDtypeStruct((NW, N), jnp.float32),
    mesh=plsc.VectorSubcoreMesh(**_MESH),
    compiler_params=pltpu.CompilerParams(needs_layout_passes=False),
    scratch_types=[
        pltpu.VMEM((N,), jnp.float32),
        pltpu.VMEM((N,), jnp.float32),
        pltpu.VMEM((CH,), jnp.int32),
        pltpu.VMEM((CH,), jnp.int32),
    ],
)(_sc_scatter1)


def _sc_scatter8(gt_hbm, src_hbm, dst_hbm, out_hbm, g_v, acc_v, src_v, dst_v):
    cid = lax.axis_index("c")
    sid = lax.axis_index("s")
    wid = sid * 2 + cid
    base = wid * EW
    zeros = jnp.zeros((16,), jnp.float32)

    for hp in range(2):
        pltpu.sync_copy(gt_hbm.at[pl.ds(hp * 4, 4)], g_v)

        def zbody(i, carry):
            off = pl.multiple_of(i * 16, 16)
            for f in range(4):
                acc_v[f, pl.ds(off, 16)] = zeros
            return carry

        lax.fori_loop(0, N // 16, zbody, 0)

        def chunk(ci, carry):
            pltpu.sync_copy(src_hbm.at[pl.ds(base + ci * CH, CH)], src_v)
            pltpu.sync_copy(dst_hbm.at[pl.ds(base + ci * CH, CH)], dst_v)

            def vec(i, c2):
                off = pl.multiple_of(i * 16, 16)
                si = src_v[pl.ds(off, 16)]
                di = dst_v[pl.ds(off, 16)]
                for f in range(4):
                    fv = jnp.full((16,), f, jnp.int32)
                    vals = plsc.load_gather(g_v, [fv, si])
                    plsc.addupdate_scatter(acc_v, [fv, di], vals)
                return c2

            lax.fori_loop(0, CH // 16, vec, 0)
            return carry

        lax.fori_loop(0, EW // CH, chunk, 0)
        pltpu.sync_copy(acc_v, out_hbm.at[wid, pl.ds(hp * 4, 4)])


_scatter8 = functools.partial(
    pl.kernel,
    out_type=jax.ShapeDtypeStruct((NW, N_OUT, N), jnp.float32),
    mesh=plsc.VectorSubcoreMesh(**_MESH),
    compiler_params=pltpu.CompilerParams(needs_layout_passes=False),
    scratch_types=[
        pltpu.VMEM((4, N), jnp.float32),
        pltpu.VMEM((4, N), jnp.float32),
        pltpu.VMEM((CH,), jnp.int32),
        pltpu.VMEM((CH,), jnp.int32),
    ],
)(_sc_scatter8)


def _tc_dense(parts_ref, h_ref, w1rel_ref, w1root_ref, b1_ref, w2relT_ref,
              w2rootT_ref, gt_ref, rt_ref):
    s = jnp.sum(parts_ref[...], axis=0, keepdims=True)
    f = jnp.maximum(
        w1rel_ref[...] * s + w1root_ref[...] * h_ref[...] + b1_ref[...], 0.0)
    gt_ref[...] = jnp.dot(w2relT_ref[...], f,
                          preferred_element_type=jnp.float32)
    rt_ref[...] = jnp.dot(w2rootT_ref[...], f,
                          preferred_element_type=jnp.float32)


def _tc_lstm(parts_ref, rt_ref, b2_ref, wih_ref, bias_ref, whhhi_ref,
             whhlo_ref, wout_ref, wx_ref, xlast_ref, out_ref,
             p_ref, outs_ref):
    tt = jnp.sum(parts_ref[...], axis=0)
    ht = tt + rt_ref[...] + b2_ref[...]
    dn = (((0,), (0,)), ((), ()))
    p_ref[...] = lax.dot_general(
        ht, wih_ref[...], dn, preferred_element_type=jnp.float32,
    ) + bias_ref[...]

    whh_hi = whhhi_ref[...]
    whh_lo = whhlo_ref[...]
    lane = lax.broadcasted_iota(jnp.int32, (1, 2 * HID), 1)
    lmask = lane < HID

    def body(t, carry):
        h, c = carry
        hh = h[:, :HID]
        h_hi = hh.astype(jnp.bfloat16)
        h_lo = (hh - h_hi.astype(jnp.float32)).astype(jnp.bfloat16)
        u = (jnp.dot(h_hi, whh_hi, preferred_element_type=jnp.float32)
             + jnp.dot(h_lo, whh_hi, preferred_element_type=jnp.float32)
             + jnp.dot(h_hi, whh_lo, preferred_element_type=jnp.float32))
        z = u + p_ref[pl.ds(t, 1), :]
        tz = jnp.tanh(z)
        sz = 0.5 * tz + 0.5
        t1 = tz[:, 2 * HD2:3 * HD2]
        t1s = tz[:, 3 * HD2:4 * HD2]
        s0 = sz[:, 0 * HD2:1 * HD2]
        s0s = sz[:, 1 * HD2:2 * HD2]
        s1 = sz[:, 2 * HD2:3 * HD2]
        s1s = sz[:, 3 * HD2:4 * HD2]
        c2 = s0 * jnp.where(lmask, t1, c) + s0s * jnp.where(lmask, c, t1s)
        so = jnp.where(lmask, s1s, s1)
        h2 = so * jnp.tanh(c2)
        outs_ref[pl.ds(t, 1), :] = h2[:, :HID]
        return (h2, c2)

    h0 = jnp.zeros((1, 2 * HID), jnp.float32)
    lax.fori_loop(0, N, body, (h0, h0), unroll=4)

    const = jnp.sum(wx_ref[...] * xlast_ref[...])
    out_ref[...] = jnp.dot(outs_ref[...], wout_ref[...],
                           preferred_element_type=jnp.float32) + const


def kernel(x, edge_index, edge_attr, weather, time_encoding,
           conv1_W_rel, conv1_b_rel, conv1_W_root,
           conv2_W_rel, conv2_b_rel, conv2_W_root,
           W_ih, W_hh, b_ih, b_hh, W_lin, b_lin):
    del edge_attr
    h1d = x[0, -1, :]
    src = edge_index[0]
    dst = edge_index[1]

    parts_s = _scatter1(h1d, src, dst)

    tc_dense = pl.pallas_call(
        _tc_dense,
        out_shape=(
            jax.ShapeDtypeStruct((N_OUT, N), jnp.float32),
            jax.ShapeDtypeStruct((N_OUT, N), jnp.float32),
        ),
    )
    gt, rt = tc_dense(
        parts_s,
        h1d.reshape(1, N),
        conv1_W_rel.reshape(G_HID, 1),
        conv1_W_root.reshape(G_HID, 1),
        conv1_b_rel.reshape(G_HID, 1),
        conv2_W_rel.T,
        conv2_W_root.T,
    )

    parts_t = _scatter8(gt, src, dst)

    xlast = jnp.concatenate(
        [weather[0, -1, :], time_encoding[0, -1, :],
         jnp.ones((1,), jnp.float32)]).reshape(1, 15)
    wx = jnp.concatenate([W_lin[HID:, 0], b_lin]).reshape(1, 15)

    ii = jnp.arange(HID)
    blk = [ii, HID + ii, HID + ii, ii,
           2 * HID + ii, 3 * HID + ii, 3 * HID + ii, 2 * HID + ii]
    perm = jnp.concatenate(blk)
    is_g = (perm >= 2 * HID) & (perm < 3 * HID)
    scale = jnp.where(is_g, 1.0, 0.5).astype(jnp.float32)

    wih512 = (W_ih.T)[:, perm] * scale
    bias512 = ((b_ih + b_hh)[perm] * scale).reshape(1, 4 * HD2)
    whh512 = (W_hh.T)[:, perm] * scale
    whh_hi = whh512.astype(jnp.bfloat16)
    whh_lo = (whh512 - whh_hi.astype(jnp.float32)).astype(jnp.bfloat16)

    tc_lstm = pl.pallas_call(
        _tc_lstm,
        out_shape=jax.ShapeDtypeStruct((N, 1), jnp.float32),
        scratch_shapes=[
            pltpu.VMEM((N, 4 * HD2), jnp.float32),
            pltpu.VMEM((N, HID), jnp.float32),
        ],
    )
    pred = tc_lstm(
        parts_t,
        rt,
        conv2_b_rel.reshape(N_OUT, 1),
        wih512,
        bias512,
        whh_hi,
        whh_lo,
        W_lin[:HID],
        wx,
        xlast,
    )
    return pred.reshape(1, N, 1)


# speedup vs baseline: 1.2109x; 1.2109x over previous
import functools

import jax
import jax.numpy as jnp
from jax import lax
from jax.experimental import pallas as pl
from jax.experimental.pallas import tpu as pltpu
from jax.experimental.pallas import tpu_sc as plsc

N = 10000
E = 640000
NW = 32
EW = E // NW
CH = 10000
G_HID = 32
N_OUT = 8
HID = 64
HD2 = 2 * HID

_MESH = dict(core_axis_name="c", subcore_axis_name="s", num_cores=2,
             num_subcores=16)


def _sc_scatter1(h_hbm, src_hbm, dst_hbm, out_hbm, h_v, acc_v, src_v, dst_v):
    cid = lax.axis_index("c")
    sid = lax.axis_index("s")
    wid = sid * 2 + cid
    base = wid * EW
    pltpu.sync_copy(h_hbm, h_v)

    zeros = jnp.zeros((16,), jnp.float32)

    def zbody(i, carry):
        acc_v[pl.ds(pl.multiple_of(i * 16, 16), 16)] = zeros
        return carry

    lax.fori_loop(0, N // 16, zbody, 0)

    def chunk(ci, carry):
        pltpu.sync_copy(src_hbm.at[pl.ds(base + ci * CH, CH)], src_v)
        pltpu.sync_copy(dst_hbm.at[pl.ds(base + ci * CH, CH)], dst_v)

        def vec(i, c2):
            off = pl.multiple_of(i * 16, 16)
            si = src_v[pl.ds(off, 16)]
            di = dst_v[pl.ds(off, 16)]
            vals = plsc.load_gather(h_v, [si])
            plsc.addupdate_scatter(acc_v, [di], vals)
            return c2

        lax.fori_loop(0, CH // 16, vec, 0)
        return carry

    lax.fori_loop(0, EW // CH, chunk, 0)
    pltpu.sync_copy(acc_v, out_hbm.at[wid])


_scatter1 = functools.partial(
    pl.kernel,
    out_type=jax.ShapeDtypeStruct((NW, N), jnp.float32),
    mesh=plsc.VectorSubcoreMesh(**_MESH),
    compiler_params=pltpu.CompilerParams(needs_layout_passes=False),
    scratch_types=[
        pltpu.VMEM((N,), jnp.float32),
        pltpu.VMEM((N,), jnp.float32),
        pltpu.VMEM((CH,), jnp.int32),
        pltpu.VMEM((CH,), jnp.int32),
    ],
)(_sc_scatter1)


def _sc_scatter8(gt_hbm, src_hbm, dst_hbm, out_hbm, g_v, acc_v, src_v, dst_v):
    cid = lax.axis_index("c")
    sid = lax.axis_index("s")
    wid = sid * 2 + cid
    base = wid * EW
    zeros = jnp.zeros((16,), jnp.float32)

    for hp in range(2):
        pltpu.sync_copy(gt_hbm.at[pl.ds(hp * 4, 4)], g_v)

        def zbody(i, carry):
            off = pl.multiple_of(i * 16, 16)
            for f in range(4):
                acc_v[f, pl.ds(off, 16)] = zeros
            return carry

        lax.fori_loop(0, N // 16, zbody, 0)

        def chunk(ci, carry):
            pltpu.sync_copy(src_hbm.at[pl.ds(base + ci * CH, CH)], src_v)
            pltpu.sync_copy(dst_hbm.at[pl.ds(base + ci * CH, CH)], dst_v)

            def vec(i, c2):
                off = pl.multiple_of(i * 16, 16)
                si = src_v[pl.ds(off, 16)]
                di = dst_v[pl.ds(off, 16)]
                for f in range(4):
                    fv = jnp.full((16,), f, jnp.int32)
                    vals = plsc.load_gather(g_v, [fv, si])
                    plsc.addupdate_scatter(acc_v, [fv, di], vals)
                return c2

            lax.fori_loop(0, CH // 16, vec, 0)
            return carry

        lax.fori_loop(0, EW // CH, chunk, 0)
        pltpu.sync_copy(acc_v, out_hbm.at[wid, pl.ds(hp * 4, 4)])


_scatter8 = functools.partial(
    pl.kernel,
    out_type=jax.ShapeDtypeStruct((NW, N_OUT, N), jnp.float32),
    mesh=plsc.VectorSubcoreMesh(**_MESH),
    compiler_params=pltpu.CompilerParams(needs_layout_passes=False),
    scratch_types=[
        pltpu.VMEM((4, N), jnp.float32),
        pltpu.VMEM((4, N), jnp.float32),
        pltpu.VMEM((CH,), jnp.int32),
        pltpu.VMEM((CH,), jnp.int32),
    ],
)(_sc_scatter8)


def _tc_dense(parts_ref, h_ref, w1rel_ref, w1root_ref, b1_ref, w2relT_ref,
              w2rootT_ref, gt_ref, rt_ref):
    s = jnp.sum(parts_ref[...], axis=0, keepdims=True)
    f = jnp.maximum(
        w1rel_ref[...] * s + w1root_ref[...] * h_ref[...] + b1_ref[...], 0.0)
    gt_ref[...] = jnp.dot(w2relT_ref[...], f,
                          preferred_element_type=jnp.float32)
    rt_ref[...] = jnp.dot(w2rootT_ref[...], f,
                          preferred_element_type=jnp.float32)


def _tc_lstm(parts_ref, rt_ref, b2_ref, wih_ref, bias_ref, whh_ref,
             wout_ref, wx_ref, xlast_ref, out_ref,
             p_ref, outs_ref):
    tt = jnp.sum(parts_ref[...], axis=0)
    ht = tt + rt_ref[...] + b2_ref[...]
    dn = (((0,), (0,)), ((), ()))
    p_ref[...] = lax.dot_general(
        ht, wih_ref[...], dn, preferred_element_type=jnp.float32,
    ) + bias_ref[...]

    whh = whh_ref[...]
    lane = lax.broadcasted_iota(jnp.int32, (1, 2 * HID), 1)
    lmask = lane < HID

    def body(t, carry):
        h, c = carry
        u = jnp.dot(h[:, :HID], whh, preferred_element_type=jnp.float32)
        z = u + p_ref[pl.ds(t, 1), :]
        tz = jnp.tanh(z)
        sz = 0.5 * tz + 0.5
        t1 = tz[:, 2 * HD2:3 * HD2]
        t1s = tz[:, 3 * HD2:4 * HD2]
        s0 = sz[:, 0 * HD2:1 * HD2]
        s0s = sz[:, 1 * HD2:2 * HD2]
        s1 = sz[:, 2 * HD2:3 * HD2]
        s1s = sz[:, 3 * HD2:4 * HD2]
        c2 = s0 * jnp.where(lmask, t1, c) + s0s * jnp.where(lmask, c, t1s)
        so = jnp.where(lmask, s1s, s1)
        h2 = so * jnp.tanh(c2)
        outs_ref[pl.ds(t, 1), :] = h2[:, :HID]
        return (h2, c2)

    h0 = jnp.zeros((1, 2 * HID), jnp.float32)
    lax.fori_loop(0, N, body, (h0, h0), unroll=8)

    const = jnp.sum(wx_ref[...] * xlast_ref[...])
    out_ref[...] = jnp.dot(outs_ref[...], wout_ref[...],
                           preferred_element_type=jnp.float32) + const


def kernel(x, edge_index, edge_attr, weather, time_encoding,
           conv1_W_rel, conv1_b_rel, conv1_W_root,
           conv2_W_rel, conv2_b_rel, conv2_W_root,
           W_ih, W_hh, b_ih, b_hh, W_lin, b_lin):
    del edge_attr
    h1d = x[0, -1, :]
    src = edge_index[0]
    dst = edge_index[1]

    parts_s = _scatter1(h1d, src, dst)

    tc_dense = pl.pallas_call(
        _tc_dense,
        out_shape=(
            jax.ShapeDtypeStruct((N_OUT, N), jnp.float32),
            jax.ShapeDtypeStruct((N_OUT, N), jnp.float32),
        ),
    )
    gt, rt = tc_dense(
        parts_s,
        h1d.reshape(1, N),
        conv1_W_rel.reshape(G_HID, 1),
        conv1_W_root.reshape(G_HID, 1),
        conv1_b_rel.reshape(G_HID, 1),
        conv2_W_rel.T,
        conv2_W_root.T,
    )

    parts_t = _scatter8(gt, src, dst)

    xlast = jnp.concatenate(
        [weather[0, -1, :], time_encoding[0, -1, :],
         jnp.ones((1,), jnp.float32)]).reshape(1, 15)
    wx = jnp.concatenate([W_lin[HID:, 0], b_lin]).reshape(1, 15)

    ii = jnp.arange(HID)
    blk = [ii, HID + ii, HID + ii, ii,
           2 * HID + ii, 3 * HID + ii, 3 * HID + ii, 2 * HID + ii]
    perm = jnp.concatenate(blk)
    is_g = (perm >= 2 * HID) & (perm < 3 * HID)
    scale = jnp.where(is_g, 1.0, 0.5).astype(jnp.float32)

    wih512 = (W_ih.T)[:, perm] * scale
    bias512 = ((b_ih + b_hh)[perm] * scale).reshape(1, 4 * HD2)
    whh512 = (W_hh.T)[:, perm] * scale

    tc_lstm = pl.pallas_call(
        _tc_lstm,
        out_shape=jax.ShapeDtypeStruct((N, 1), jnp.float32),
        scratch_shapes=[
            pltpu.VMEM((N, 4 * HD2), jnp.float32),
            pltpu.VMEM((N, HID), jnp.float32),
        ],
    )
    pred = tc_lstm(
        parts_t,
        rt,
        conv2_b_rel.reshape(N_OUT, 1),
        wih512,
        bias512,
        whh512,
        W_lin[:HID],
        wx,
        xlast,
    )
    return pred.reshape(1, N, 1)
